# baseline (speedup 1.0000x reference)
import functools

import jax
import jax.numpy as jnp
from jax import lax
from jax.experimental import pallas as pl
from jax.experimental.pallas import tpu as pltpu
from jax.experimental.pallas import tpu_sc as plsc

N = 8192
K = 32
KP1 = 33
NSIG = 64
R = 256
RE = 512
NG = 64
GW = N // NG
SLOTS = 24
SIG0, SIG1 = 0.1, 5.0
BIGF = 3.0e38
BIGI = 1 << 30
NW = 32
RPW = N // NW


def _d2_thr_block(xp_ref, xt_ref, d2_ref, thr_ref):
    xb = xp_ref[...]
    xt = xt_ref[...]
    dot = jnp.dot(xb, xt, preferred_element_type=jnp.float32)
    x2r = jnp.sum(xb * xb, axis=1, keepdims=True)
    x2c = jnp.sum(xt * xt, axis=0, keepdims=True)
    d2 = jnp.maximum(x2r + x2c - 2.0 * dot, 0.0)
    colf = lax.broadcasted_iota(jnp.int32, (R, N), 1).astype(jnp.float32)
    d2 = jnp.where(d2 == 0.0, (colf - 8192.0) * (2.0 ** -100), d2)
    d2_ref[...] = d2

    gmins = [jnp.min(lax.slice(d2, (0, g * GW), (R, (g + 1) * GW)),
                     axis=1, keepdims=True) for g in range(NG)]
    a = jnp.concatenate(gmins, axis=1)
    lane = lax.broadcasted_iota(jnp.int32, (R, NG), 1)

    def step(t, carry):
        a, _ = carry
        m = jnp.min(a, axis=1, keepdims=True)
        cand = jnp.where(a == m, lane, BIGI)
        win = jnp.min(cand, axis=1, keepdims=True)
        a = jnp.where(lane == win, BIGF, a)
        return a, m

    _, thr = lax.fori_loop(0, KP1, step, (a, jnp.zeros((R, 1), jnp.float32)))
    thr_ref[...] = jnp.broadcast_to(thr, (R, 16))


def _merge48_16(l0, l1, l2, c0, c1, c2, vs, cs):
    rv = lax.rev(vs, (0,))
    rc = lax.rev(cs, (0,))
    m = l2 <= rv
    nl2 = jnp.where(m, l2, rv)
    nc2 = jnp.where(m, c2, rc)
    nl2, nc2 = plsc.sort_key_val(nl2, nc2)
    rv2 = lax.rev(nl2, (0,))
    rc2 = lax.rev(nc2, (0,))
    m2 = l1 <= rv2
    t1 = jnp.where(m2, l1, rv2)
    tc1 = jnp.where(m2, c1, rc2)
    h1 = jnp.where(m2, rv2, l1)
    hc1 = jnp.where(m2, rc2, c1)
    t1, tc1 = plsc.sort_key_val(t1, tc1)
    h1, hc1 = plsc.sort_key_val(h1, hc1)
    rv3 = lax.rev(t1, (0,))
    rc3 = lax.rev(tc1, (0,))
    m3 = l0 <= rv3
    lo = jnp.where(m3, l0, rv3)
    lc = jnp.where(m3, c0, rc3)
    hi = jnp.where(m3, rv3, l0)
    hc = jnp.where(m3, rc3, c0)
    lo, lc = plsc.sort_key_val(lo, lc)
    hi, hc = plsc.sort_key_val(hi, hc)
    return lo, hi, h1, lc, hc, hc1


def _sc_topk(d2_hbm, thr_hbm, svals_hbm, scols_hbm, rowbuf, thrbuf, vbuf,
             cbuf, ovbuf, ocbuf, rsem, osem, csem):
    wid = lax.axis_index("s") * 2 + lax.axis_index("c")
    base = wid * RPW
    pltpu.sync_copy(thr_hbm.at[pl.ds(base, RPW), :], thrbuf)

    lane = jnp.arange(16, dtype=jnp.int32)
    lanef = lane.astype(jnp.float32)
    infv = jnp.full((16,), BIGF, jnp.float32) + 0.0 * lanef
    zero16 = jnp.zeros((16,), jnp.int32)

    def in_copy(r):
        return pltpu.make_async_copy(
            d2_hbm.at[pl.ds(base + r, 1), :],
            rowbuf.at[pl.ds(lax.rem(r, 2), 1), :],
            rsem.at[lax.rem(r, 2)])

    def out_copies(r):
        slot = lax.rem(r, 4)
        return (pltpu.make_async_copy(ovbuf.at[pl.ds(slot, 1), :],
                                      svals_hbm.at[pl.ds(base + r, 1), :],
                                      osem.at[slot]),
                pltpu.make_async_copy(ocbuf.at[pl.ds(slot, 1), :],
                                      scols_hbm.at[pl.ds(base + r, 1), :],
                                      csem.at[slot]))

    in_copy(0).start()

    def row_body(r, _):
        @pl.when(r + 1 < RPW)
        def _():
            in_copy(r + 1).start()

        in_copy(r).wait()
        rslot = lax.rem(r, 2)
        tv = thrbuf[r, pl.ds(0, 16)]
        for s in range(SLOTS):
            vbuf[pl.ds(s * 16, 16)] = infv

        @plsc.parallel_loop(0, N // 16, unroll=8, carry=(lane, lane))
        def fin(j, carry):
            addr, colv = carry
            v = rowbuf[rslot, pl.ds(j * 16, 16)]
            msk = (v <= tv) & (addr < 16 * SLOTS)
            plsc.store_scatter(vbuf, [addr], v, mask=msk)
            plsc.store_scatter(cbuf, [addr], colv, mask=msk)
            return addr + (msk.astype(jnp.int32) << 4), colv + 16

        addr_fin, _ = fin
        maxc = jnp.max(addr_fin >> 4)

        def mbody(s, carry):
            l0, l1, l2, c0, c1, c2 = carry
            vs = vbuf[pl.ds(s * 16, 16)]
            cs = cbuf[pl.ds(s * 16, 16)]
            vs, cs = plsc.sort_key_val(vs, cs)
            return _merge48_16(l0, l1, l2, c0, c1, c2, vs, cs)

        init = (infv, infv, infv, zero16, zero16, zero16)
        l0, l1, l2, c0, c1, c2 = lax.fori_loop(0, maxc, mbody, init)

        @pl.when(r >= 4)
        def _():
            ov, oc = out_copies(r - 4)
            ov.wait()
            oc.wait()

        oslot = lax.rem(r, 4)
        ovbuf[oslot, pl.ds(0, 16)] = l0
        ovbuf[oslot, pl.ds(16, 16)] = l1
        ovbuf[oslot, pl.ds(32, 16)] = l2
        ocbuf[oslot, pl.ds(0, 16)] = c0
        ocbuf[oslot, pl.ds(16, 16)] = c1
        ocbuf[oslot, pl.ds(32, 16)] = c2
        ov, oc = out_copies(r)
        ov.start()
        oc.start()
        return 0

    lax.fori_loop(0, RPW, row_body, 0)
    for t in range(4):
        ov, oc = out_copies(RPW - 4 + t)
        ov.wait()
        oc.wait()


def _emb_block(sv_ref, emb_ref):
    vals_k = lax.slice(sv_ref[...], (0, 1), (RE, KP1))
    sig_i = lax.broadcasted_iota(jnp.int32, (1, 1, NSIG), 2).astype(jnp.float32)
    sig = SIG0 + sig_i * ((SIG1 - SIG0) / (NSIG - 1))
    inv = 0.5 / (sig * sig)
    e3 = jnp.exp(-vals_k[:, :, None] * inv)
    emb_ref[...] = e3.reshape(1, RE * K, NSIG)


@jax.jit
def kernel(input_coord):
    x = input_coord.astype(jnp.float32)
    xp = jnp.pad(x, ((0, 0), (0, 5)))
    xt = xp.T

    d2, thr = pl.pallas_call(
        _d2_thr_block,
        grid=(N // R,),
        in_specs=[
            pl.BlockSpec((R, 8), lambda b: (b, 0)),
            pl.BlockSpec((8, N), lambda b: (0, 0)),
        ],
        out_specs=[
            pl.BlockSpec((R, N), lambda b: (b, 0)),
            pl.BlockSpec((R, 16), lambda b: (b, 0)),
        ],
        out_shape=[
            jax.ShapeDtypeStruct((N, N), jnp.float32),
            jax.ShapeDtypeStruct((N, 16), jnp.float32),
        ],
    )(xp, xt)

    mesh = plsc.VectorSubcoreMesh(core_axis_name="c", subcore_axis_name="s")
    sc_call = functools.partial(
        pl.kernel,
        mesh=mesh,
        compiler_params=pltpu.CompilerParams(needs_layout_passes=False),
        out_type=[
            jax.ShapeDtypeStruct((N, 48), jnp.float32),
            jax.ShapeDtypeStruct((N, 48), jnp.int32),
        ],
        scratch_types=[
            pltpu.VMEM((2, N), jnp.float32),
            pltpu.VMEM((RPW, 16), jnp.float32),
            pltpu.VMEM((16 * SLOTS,), jnp.float32),
            pltpu.VMEM((16 * SLOTS,), jnp.int32),
            pltpu.VMEM((4, 48), jnp.float32),
            pltpu.VMEM((4, 48), jnp.int32),
            pltpu.SemaphoreType.DMA((2,)),
            pltpu.SemaphoreType.DMA((4,)),
            pltpu.SemaphoreType.DMA((4,)),
        ],
    )(_sc_topk)
    svals, scols = sc_call(d2, thr)

    emb = pl.pallas_call(
        _emb_block,
        grid=(N // RE,),
        in_specs=[pl.BlockSpec((RE, 48), lambda b: (b, 0))],
        out_specs=pl.BlockSpec((1, RE * K, NSIG), lambda b: (0, b, 0)),
        out_shape=jax.ShapeDtypeStruct((1, N * K, NSIG), jnp.float32),
    )(svals)

    col = lax.slice(scols, (0, 1), (N, KP1)).reshape(N * K)
    row = jnp.repeat(jnp.arange(N, dtype=jnp.int32), K)
    pairs = jnp.stack([row, col], axis=1).astype(jnp.int16)
    return emb, pairs


# speedup vs baseline: 1.0004x; 1.0004x over previous
import functools

import jax
import jax.numpy as jnp
from jax import lax
from jax.experimental import pallas as pl
from jax.experimental.pallas import tpu as pltpu
from jax.experimental.pallas import tpu_sc as plsc

N = 8192
K = 32
KP1 = 33
NSIG = 64
R = 256
RE = 512
NG = 64
GW = N // NG
SLOTS = 24
SIG0, SIG1 = 0.1, 5.0
BIGF = 3.0e38
BIGI = 1 << 30
NW = 32
RPW = N // NW


def _d2_thr_block(xp_ref, xt_ref, d2_ref, thr_ref):
    xb = xp_ref[...]
    xt = xt_ref[...]
    dot = jnp.dot(xb, xt, preferred_element_type=jnp.float32)
    x2r = jnp.sum(xb * xb, axis=1, keepdims=True)
    x2c = jnp.sum(xt * xt, axis=0, keepdims=True)
    d2 = jnp.maximum(x2r + x2c - 2.0 * dot, 0.0)
    colf = lax.broadcasted_iota(jnp.int32, (R, N), 1).astype(jnp.float32)
    d2 = jnp.where(d2 == 0.0, (colf - 8192.0) * (2.0 ** -100), d2)
    d2_ref[...] = d2

    gmins = [jnp.min(lax.slice(d2, (0, g * GW), (R, (g + 1) * GW)),
                     axis=1, keepdims=True) for g in range(NG)]
    a = jnp.concatenate(gmins, axis=1)
    lane = lax.broadcasted_iota(jnp.int32, (R, NG), 1)

    def step(t, carry):
        a, _ = carry
        m = jnp.min(a, axis=1, keepdims=True)
        cand = jnp.where(a == m, lane, BIGI)
        win = jnp.min(cand, axis=1, keepdims=True)
        a = jnp.where(lane == win, BIGF, a)
        return a, m

    _, thr = lax.fori_loop(0, KP1, step, (a, jnp.zeros((R, 1), jnp.float32)))
    thr_ref[...] = jnp.broadcast_to(thr, (R, 16))


def _merge48_16(l0, l1, l2, c0, c1, c2, vs, cs):
    rv = lax.rev(vs, (0,))
    rc = lax.rev(cs, (0,))
    m = l2 <= rv
    nl2 = jnp.where(m, l2, rv)
    nc2 = jnp.where(m, c2, rc)
    nl2, nc2 = plsc.sort_key_val(nl2, nc2)
    rv2 = lax.rev(nl2, (0,))
    rc2 = lax.rev(nc2, (0,))
    m2 = l1 <= rv2
    t1 = jnp.where(m2, l1, rv2)
    tc1 = jnp.where(m2, c1, rc2)
    h1 = jnp.where(m2, rv2, l1)
    hc1 = jnp.where(m2, rc2, c1)
    t1, tc1 = plsc.sort_key_val(t1, tc1)
    h1, hc1 = plsc.sort_key_val(h1, hc1)
    rv3 = lax.rev(t1, (0,))
    rc3 = lax.rev(tc1, (0,))
    m3 = l0 <= rv3
    lo = jnp.where(m3, l0, rv3)
    lc = jnp.where(m3, c0, rc3)
    hi = jnp.where(m3, rv3, l0)
    hc = jnp.where(m3, rc3, c0)
    lo, lc = plsc.sort_key_val(lo, lc)
    hi, hc = plsc.sort_key_val(hi, hc)
    return lo, hi, h1, lc, hc, hc1


def _sc_topk(d2_hbm, thr_hbm, svals_hbm, scols_hbm, rowbuf, thrbuf, vbuf,
             cbuf, ovbuf, ocbuf, rsem, osem, csem):
    wid = lax.axis_index("s") * 2 + lax.axis_index("c")
    base = wid * RPW
    pltpu.sync_copy(thr_hbm.at[pl.ds(base, RPW), :], thrbuf)

    lane = jnp.arange(16, dtype=jnp.int32)
    lanef = lane.astype(jnp.float32)
    infv = jnp.full((16,), BIGF, jnp.float32) + 0.0 * lanef
    zero16 = jnp.zeros((16,), jnp.int32)

    def in_copy(r):
        return pltpu.make_async_copy(
            d2_hbm.at[pl.ds(base + r, 1), :],
            rowbuf.at[pl.ds(lax.rem(r, 2), 1), :],
            rsem.at[lax.rem(r, 2)])

    def out_copies(r):
        slot = lax.rem(r, 4)
        return (pltpu.make_async_copy(ovbuf.at[pl.ds(slot, 1), :],
                                      svals_hbm.at[pl.ds(base + r, 1), :],
                                      osem.at[slot]),
                pltpu.make_async_copy(ocbuf.at[pl.ds(slot, 1), :],
                                      scols_hbm.at[pl.ds(base + r, 1), :],
                                      csem.at[slot]))

    in_copy(0).start()

    def row_body(r, _):
        @pl.when(r + 1 < RPW)
        def _():
            in_copy(r + 1).start()

        in_copy(r).wait()
        rslot = lax.rem(r, 2)
        tv = thrbuf[r, pl.ds(0, 16)]
        for s in range(SLOTS):
            vbuf[pl.ds(s * 16, 16)] = infv

        @plsc.parallel_loop(0, N // 16, unroll=8, carry=zero16)
        def cnt_fin(j, cnt):
            v = rowbuf[rslot, pl.ds(j * 16, 16)]
            msk = (v <= tv) & (cnt < SLOTS)
            addr = cnt * 16 + lane
            colv = jnp.full((16,), j * 16, jnp.int32) + lane
            plsc.store_scatter(vbuf, [addr], v, mask=msk)
            plsc.store_scatter(cbuf, [addr], colv, mask=msk)
            return cnt + msk.astype(jnp.int32)

        maxc = jnp.max(cnt_fin)

        def mbody(s, carry):
            l0, l1, l2, c0, c1, c2 = carry
            vs = vbuf[pl.ds(s * 16, 16)]
            cs = cbuf[pl.ds(s * 16, 16)]
            vs, cs = plsc.sort_key_val(vs, cs)
            return _merge48_16(l0, l1, l2, c0, c1, c2, vs, cs)

        init = (infv, infv, infv, zero16, zero16, zero16)
        l0, l1, l2, c0, c1, c2 = lax.fori_loop(0, maxc, mbody, init)

        @pl.when(r >= 4)
        def _():
            ov, oc = out_copies(r - 4)
            ov.wait()
            oc.wait()

        oslot = lax.rem(r, 4)
        ovbuf[oslot, pl.ds(0, 16)] = l0
        ovbuf[oslot, pl.ds(16, 16)] = l1
        ovbuf[oslot, pl.ds(32, 16)] = l2
        ocbuf[oslot, pl.ds(0, 16)] = c0
        ocbuf[oslot, pl.ds(16, 16)] = c1
        ocbuf[oslot, pl.ds(32, 16)] = c2
        ov, oc = out_copies(r)
        ov.start()
        oc.start()
        return 0

    lax.fori_loop(0, RPW, row_body, 0)
    for t in range(4):
        ov, oc = out_copies(RPW - 4 + t)
        ov.wait()
        oc.wait()


def _emb_block(sv_ref, emb_ref):
    vals_k = lax.slice(sv_ref[...], (0, 1), (RE, KP1))
    sig_i = lax.broadcasted_iota(jnp.int32, (1, 1, NSIG), 2).astype(jnp.float32)
    sig = SIG0 + sig_i * ((SIG1 - SIG0) / (NSIG - 1))
    inv = 0.5 / (sig * sig)
    e3 = jnp.exp(-vals_k[:, :, None] * inv)
    emb_ref[...] = e3.reshape(1, RE * K, NSIG)


@jax.jit
def kernel(input_coord):
    x = input_coord.astype(jnp.float32)
    xp = jnp.pad(x, ((0, 0), (0, 5)))
    xt = xp.T

    d2, thr = pl.pallas_call(
        _d2_thr_block,
        grid=(N // R,),
        in_specs=[
            pl.BlockSpec((R, 8), lambda b: (b, 0)),
            pl.BlockSpec((8, N), lambda b: (0, 0)),
        ],
        out_specs=[
            pl.BlockSpec((R, N), lambda b: (b, 0)),
            pl.BlockSpec((R, 16), lambda b: (b, 0)),
        ],
        out_shape=[
            jax.ShapeDtypeStruct((N, N), jnp.float32),
            jax.ShapeDtypeStruct((N, 16), jnp.float32),
        ],
    )(xp, xt)

    mesh = plsc.VectorSubcoreMesh(core_axis_name="c", subcore_axis_name="s")
    sc_call = functools.partial(
        pl.kernel,
        mesh=mesh,
        compiler_params=pltpu.CompilerParams(needs_layout_passes=False),
        out_type=[
            jax.ShapeDtypeStruct((N, 48), jnp.float32),
            jax.ShapeDtypeStruct((N, 48), jnp.int32),
        ],
        scratch_types=[
            pltpu.VMEM((2, N), jnp.float32),
            pltpu.VMEM((RPW, 16), jnp.float32),
            pltpu.VMEM((16 * SLOTS,), jnp.float32),
            pltpu.VMEM((16 * SLOTS,), jnp.int32),
            pltpu.VMEM((4, 48), jnp.float32),
            pltpu.VMEM((4, 48), jnp.int32),
            pltpu.SemaphoreType.DMA((2,)),
            pltpu.SemaphoreType.DMA((4,)),
            pltpu.SemaphoreType.DMA((4,)),
        ],
    )(_sc_topk)
    svals, scols = sc_call(d2, thr)

    emb = pl.pallas_call(
        _emb_block,
        grid=(N // RE,),
        in_specs=[pl.BlockSpec((RE, 48), lambda b: (b, 0))],
        out_specs=pl.BlockSpec((1, RE * K, NSIG), lambda b: (0, b, 0)),
        out_shape=jax.ShapeDtypeStruct((1, N * K, NSIG), jnp.float32),
    )(svals)

    col = lax.slice(scols, (0, 1), (N, KP1)).reshape(N * K)
    row = jnp.repeat(jnp.arange(N, dtype=jnp.int32), K)
    pairs = jnp.stack([row, col], axis=1).astype(jnp.int16)
    return emb, pairs


# speedup vs baseline: 1.0423x; 1.0419x over previous
import functools

import jax
import jax.numpy as jnp
from jax import lax
from jax.experimental import pallas as pl
from jax.experimental.pallas import tpu as pltpu
from jax.experimental.pallas import tpu_sc as plsc

N = 8192
K = 32
KP1 = 33
NSIG = 64
R = 256
RE = 512
NG = 64
GW = N // NG
SLOTS = 24
SIG0, SIG1 = 0.1, 5.0
BIGF = 3.0e38
BIGI = 1 << 30
NW = 32
RPW = N // NW


def _d2_thr_block(xp_ref, xt_ref, d2_ref, thr_ref):
    xb = xp_ref[...]
    xt = xt_ref[...]
    dot = jnp.dot(xb, xt, preferred_element_type=jnp.float32)
    x2r = jnp.sum(xb * xb, axis=1, keepdims=True)
    x2c = jnp.sum(xt * xt, axis=0, keepdims=True)
    d2 = jnp.maximum(x2r + x2c - 2.0 * dot, 0.0)
    colf = lax.broadcasted_iota(jnp.int32, (R, N), 1).astype(jnp.float32)
    d2 = jnp.where(d2 == 0.0, (colf - 8192.0) * (2.0 ** -100), d2)
    d2_ref[...] = d2

    gmins = [jnp.min(lax.slice(d2, (0, g * GW), (R, (g + 1) * GW)),
                     axis=1, keepdims=True) for g in range(NG)]
    a = jnp.concatenate(gmins, axis=1)
    lane = lax.broadcasted_iota(jnp.int32, (R, NG), 1)

    def step(t, carry):
        a, _ = carry
        m = jnp.min(a, axis=1, keepdims=True)
        cand = jnp.where(a == m, lane, BIGI)
        win = jnp.min(cand, axis=1, keepdims=True)
        a = jnp.where(lane == win, BIGF, a)
        return a, m

    _, thr = lax.fori_loop(0, KP1, step, (a, jnp.zeros((R, 1), jnp.float32)))
    thr_ref[...] = jnp.broadcast_to(thr, (R, 16))


def _merge48_16(l0, l1, l2, c0, c1, c2, vs, cs):
    rv = lax.rev(vs, (0,))
    rc = lax.rev(cs, (0,))
    m = l2 <= rv
    nl2 = jnp.where(m, l2, rv)
    nc2 = jnp.where(m, c2, rc)
    nl2, nc2 = plsc.sort_key_val(nl2, nc2)
    rv2 = lax.rev(nl2, (0,))
    rc2 = lax.rev(nc2, (0,))
    m2 = l1 <= rv2
    t1 = jnp.where(m2, l1, rv2)
    tc1 = jnp.where(m2, c1, rc2)
    h1 = jnp.where(m2, rv2, l1)
    hc1 = jnp.where(m2, rc2, c1)
    t1, tc1 = plsc.sort_key_val(t1, tc1)
    h1, hc1 = plsc.sort_key_val(h1, hc1)
    rv3 = lax.rev(t1, (0,))
    rc3 = lax.rev(tc1, (0,))
    m3 = l0 <= rv3
    lo = jnp.where(m3, l0, rv3)
    lc = jnp.where(m3, c0, rc3)
    hi = jnp.where(m3, rv3, l0)
    hc = jnp.where(m3, rc3, c0)
    lo, lc = plsc.sort_key_val(lo, lc)
    hi, hc = plsc.sort_key_val(hi, hc)
    return lo, hi, h1, lc, hc, hc1


def _sc_topk(d2_hbm, thr_hbm, svals_hbm, scols_hbm, rowbuf, thrbuf, vbuf,
             cbuf, ovbuf, ocbuf, rsem, osem, csem):
    wid = lax.axis_index("s") * 2 + lax.axis_index("c")
    base = wid * RPW
    pltpu.sync_copy(thr_hbm.at[pl.ds(base, RPW), :], thrbuf)

    lane = jnp.arange(16, dtype=jnp.int32)
    lanef = lane.astype(jnp.float32)
    infv = jnp.full((16,), BIGF, jnp.float32) + 0.0 * lanef
    zero16 = jnp.zeros((16,), jnp.int32)

    def in_copy(r):
        return pltpu.make_async_copy(
            d2_hbm.at[pl.ds(base + r, 1), :],
            rowbuf.at[pl.ds(lax.rem(r, 2), 1), :],
            rsem.at[lax.rem(r, 2)])

    def out_copies(r):
        slot = lax.rem(r, 4)
        return (pltpu.make_async_copy(ovbuf.at[pl.ds(slot, 1), :],
                                      svals_hbm.at[pl.ds(base + r, 1), :],
                                      osem.at[slot]),
                pltpu.make_async_copy(ocbuf.at[pl.ds(slot, 1), :],
                                      scols_hbm.at[pl.ds(base + r, 1), :],
                                      csem.at[slot]))

    in_copy(0).start()

    def row_body(r, _):
        @pl.when(r + 1 < RPW)
        def _():
            in_copy(r + 1).start()

        in_copy(r).wait()
        rslot = lax.rem(r, 2)
        tv = thrbuf[r, pl.ds(0, 16)]
        for s in range(SLOTS):
            vbuf[pl.ds(s * 16, 16)] = infv

        @plsc.parallel_loop(0, N // 16, unroll=8, carry=zero16)
        def cnt_fin(j, cnt):
            v = rowbuf[rslot, pl.ds(j * 16, 16)]
            msk = (v <= tv) & (cnt < SLOTS)
            addr = cnt * 16 + lane
            colv = jnp.full((16,), j * 16, jnp.int32) + lane
            plsc.store_scatter(vbuf, [addr], v, mask=msk)
            plsc.store_scatter(cbuf, [addr], colv, mask=msk)
            return cnt + msk.astype(jnp.int32)

        maxc = jnp.max(cnt_fin)

        def mbody(s, carry):
            l0, l1, l2, c0, c1, c2 = carry
            vs = vbuf[pl.ds(s * 16, 16)]
            cs = cbuf[pl.ds(s * 16, 16)]
            vs, cs = plsc.sort_key_val(vs, cs)
            return _merge48_16(l0, l1, l2, c0, c1, c2, vs, cs)

        init = (infv, infv, infv, zero16, zero16, zero16)
        l0, l1, l2, c0, c1, c2 = lax.fori_loop(0, maxc, mbody, init)

        @pl.when(r >= 4)
        def _():
            ov, oc = out_copies(r - 4)
            ov.wait()
            oc.wait()

        oslot = lax.rem(r, 4)
        ovbuf[oslot, pl.ds(0, 16)] = l0
        ovbuf[oslot, pl.ds(16, 16)] = l1
        ovbuf[oslot, pl.ds(32, 16)] = l2
        ocbuf[oslot, pl.ds(0, 16)] = c0
        ocbuf[oslot, pl.ds(16, 16)] = c1
        ocbuf[oslot, pl.ds(32, 16)] = c2
        ov, oc = out_copies(r)
        ov.start()
        oc.start()
        return 0

    lax.fori_loop(0, RPW, row_body, 0)
    for t in range(4):
        ov, oc = out_copies(RPW - 4 + t)
        ov.wait()
        oc.wait()


def _emb_block(sv_ref, emb_ref):
    vals_k = lax.slice(sv_ref[...], (0, 1), (RE, KP1))
    sig_i = lax.broadcasted_iota(jnp.int32, (1, 1, NSIG), 2).astype(jnp.float32)
    sig = SIG0 + sig_i * ((SIG1 - SIG0) / (NSIG - 1))
    inv = 0.5 / (sig * sig)
    emb_ref[...] = jnp.exp(-vals_k[:, :, None] * inv)


@jax.jit
def kernel(input_coord):
    x = input_coord.astype(jnp.float32)
    xp = jnp.pad(x, ((0, 0), (0, 5)))
    xt = xp.T

    d2, thr = pl.pallas_call(
        _d2_thr_block,
        grid=(N // R,),
        in_specs=[
            pl.BlockSpec((R, 8), lambda b: (b, 0)),
            pl.BlockSpec((8, N), lambda b: (0, 0)),
        ],
        out_specs=[
            pl.BlockSpec((R, N), lambda b: (b, 0)),
            pl.BlockSpec((R, 16), lambda b: (b, 0)),
        ],
        out_shape=[
            jax.ShapeDtypeStruct((N, N), jnp.float32),
            jax.ShapeDtypeStruct((N, 16), jnp.float32),
        ],
    )(xp, xt)

    mesh = plsc.VectorSubcoreMesh(core_axis_name="c", subcore_axis_name="s")
    sc_call = functools.partial(
        pl.kernel,
        mesh=mesh,
        compiler_params=pltpu.CompilerParams(needs_layout_passes=False,
                                             use_tc_tiling_on_sc=True),
        out_type=[
            jax.ShapeDtypeStruct((N, 48), jnp.float32),
            jax.ShapeDtypeStruct((N, 48), jnp.int32),
        ],
        scratch_types=[
            pltpu.VMEM((2, N), jnp.float32),
            pltpu.VMEM((RPW, 16), jnp.float32),
            pltpu.VMEM((16 * SLOTS,), jnp.float32),
            pltpu.VMEM((16 * SLOTS,), jnp.int32),
            pltpu.VMEM((4, 48), jnp.float32),
            pltpu.VMEM((4, 48), jnp.int32),
            pltpu.SemaphoreType.DMA((2,)),
            pltpu.SemaphoreType.DMA((4,)),
            pltpu.SemaphoreType.DMA((4,)),
        ],
    )(_sc_topk)
    svals, scols = sc_call(d2, thr)

    emb = pl.pallas_call(
        _emb_block,
        grid=(N // RE,),
        in_specs=[pl.BlockSpec((RE, 48), lambda b: (b, 0))],
        out_specs=pl.BlockSpec((RE, K, NSIG), lambda b: (b, 0, 0)),
        out_shape=jax.ShapeDtypeStruct((N, K, NSIG), jnp.float32),
    )(svals)

    col = lax.slice(scols, (0, 1), (N, KP1)).reshape(N * K)
    row = jnp.repeat(jnp.arange(N, dtype=jnp.int32), K)
    pairs = jnp.stack([row, col], axis=1).astype(jnp.int16)
    return emb.reshape(1, N * K, NSIG), pairs


# speedup vs baseline: 1.2456x; 1.1951x over previous
import functools

import jax
import jax.numpy as jnp
from jax import lax
from jax.experimental import pallas as pl
from jax.experimental.pallas import tpu as pltpu
from jax.experimental.pallas import tpu_sc as plsc

N = 8192
K = 32
KP1 = 33
NSIG = 64
R = 256
RE = 512
NG = 64
GW = N // NG
SLOTS = 24
SIG0, SIG1 = 0.1, 5.0
BIGF = 3.0e38
BIGI = 1 << 30
NW = 32
RPW = N // NW


def _d2_thr_block(xp_ref, xt_ref, d2_ref, gm_ref):
    xb = xp_ref[...]
    xt = xt_ref[...]
    dot = jnp.dot(xb, xt, preferred_element_type=jnp.float32)
    x2r = jnp.sum(xb * xb, axis=1, keepdims=True)
    x2c = jnp.sum(xt * xt, axis=0, keepdims=True)
    d2 = jnp.maximum(x2r + x2c - 2.0 * dot, 0.0)
    colf = lax.broadcasted_iota(jnp.int32, (R, N), 1).astype(jnp.float32)
    d2 = jnp.where(d2 == 0.0, (colf - 8192.0) * (2.0 ** -100), d2)
    d2_ref[...] = d2

    m = d2
    width = N // 2
    while width >= NG:
        m = jnp.minimum(lax.slice(m, (0, 0), (R, width)),
                        lax.slice(m, (0, width), (R, 2 * width)))
        width //= 2
    gm_ref[...] = m


def _thr_block(gm_ref, thr_ref):
    a = gm_ref[...]
    lane = lax.broadcasted_iota(jnp.int32, (N, NG), 1)

    def step(t, carry):
        a, _ = carry
        m = jnp.min(a, axis=1, keepdims=True)
        cand = jnp.where(a == m, lane, BIGI)
        win = jnp.min(cand, axis=1, keepdims=True)
        a = jnp.where(lane == win, BIGF, a)
        return a, m

    _, thr = lax.fori_loop(0, KP1, step, (a, jnp.zeros((N, 1), jnp.float32)))
    thr_ref[...] = jnp.broadcast_to(thr, (N, 16))


def _merge48_16(l0, l1, l2, c0, c1, c2, vs, cs):
    rv = lax.rev(vs, (0,))
    rc = lax.rev(cs, (0,))
    m = l2 <= rv
    nl2 = jnp.where(m, l2, rv)
    nc2 = jnp.where(m, c2, rc)
    nl2, nc2 = plsc.sort_key_val(nl2, nc2)
    rv2 = lax.rev(nl2, (0,))
    rc2 = lax.rev(nc2, (0,))
    m2 = l1 <= rv2
    t1 = jnp.where(m2, l1, rv2)
    tc1 = jnp.where(m2, c1, rc2)
    h1 = jnp.where(m2, rv2, l1)
    hc1 = jnp.where(m2, rc2, c1)
    t1, tc1 = plsc.sort_key_val(t1, tc1)
    h1, hc1 = plsc.sort_key_val(h1, hc1)
    rv3 = lax.rev(t1, (0,))
    rc3 = lax.rev(tc1, (0,))
    m3 = l0 <= rv3
    lo = jnp.where(m3, l0, rv3)
    lc = jnp.where(m3, c0, rc3)
    hi = jnp.where(m3, rv3, l0)
    hc = jnp.where(m3, rc3, c0)
    lo, lc = plsc.sort_key_val(lo, lc)
    hi, hc = plsc.sort_key_val(hi, hc)
    return lo, hi, h1, lc, hc, hc1


def _sc_topk(d2_hbm, thr_hbm, svals_hbm, scols_hbm, rowbuf, thrbuf, vbuf,
             cbuf, ovbuf, ocbuf, rsem, osem, csem):
    wid = lax.axis_index("s") * 2 + lax.axis_index("c")
    base = wid * RPW
    pltpu.sync_copy(thr_hbm.at[pl.ds(base, RPW), :], thrbuf)

    lane = jnp.arange(16, dtype=jnp.int32)
    lanef = lane.astype(jnp.float32)
    infv = jnp.full((16,), BIGF, jnp.float32) + 0.0 * lanef
    zero16 = jnp.zeros((16,), jnp.int32)

    def in_copy(r):
        return pltpu.make_async_copy(
            d2_hbm.at[pl.ds(base + r, 1), :],
            rowbuf.at[pl.ds(lax.rem(r, 2), 1), :],
            rsem.at[lax.rem(r, 2)])

    def out_copies(r):
        slot = lax.rem(r, 4)
        return (pltpu.make_async_copy(ovbuf.at[pl.ds(slot, 1), :],
                                      svals_hbm.at[pl.ds(base + r, 1), :],
                                      osem.at[slot]),
                pltpu.make_async_copy(ocbuf.at[pl.ds(slot, 1), :],
                                      scols_hbm.at[pl.ds(base + r, 1), :],
                                      csem.at[slot]))

    in_copy(0).start()

    def row_body(r, _):
        @pl.when(r + 1 < RPW)
        def _():
            in_copy(r + 1).start()

        in_copy(r).wait()
        rslot = lax.rem(r, 2)
        tv = thrbuf[r, pl.ds(0, 16)]
        for s in range(SLOTS):
            vbuf[pl.ds(s * 16, 16)] = infv

        @plsc.parallel_loop(0, N // 16, unroll=8, carry=zero16)
        def cnt_fin(j, cnt):
            v = rowbuf[rslot, pl.ds(j * 16, 16)]
            msk = (v <= tv) & (cnt < SLOTS)
            addr = cnt * 16 + lane
            colv = jnp.full((16,), j * 16, jnp.int32) + lane
            plsc.store_scatter(vbuf, [addr], v, mask=msk)
            plsc.store_scatter(cbuf, [addr], colv, mask=msk)
            return cnt + msk.astype(jnp.int32)

        maxc = jnp.max(cnt_fin)

        def mbody(s, carry):
            l0, l1, l2, c0, c1, c2 = carry
            vs = vbuf[pl.ds(s * 16, 16)]
            cs = cbuf[pl.ds(s * 16, 16)]
            vs, cs = plsc.sort_key_val(vs, cs)
            return _merge48_16(l0, l1, l2, c0, c1, c2, vs, cs)

        init = (infv, infv, infv, zero16, zero16, zero16)
        l0, l1, l2, c0, c1, c2 = lax.fori_loop(0, maxc, mbody, init)

        @pl.when(r >= 4)
        def _():
            ov, oc = out_copies(r - 4)
            ov.wait()
            oc.wait()

        oslot = lax.rem(r, 4)
        ovbuf[oslot, pl.ds(0, 16)] = l0
        ovbuf[oslot, pl.ds(16, 16)] = l1
        ovbuf[oslot, pl.ds(32, 16)] = l2
        ocbuf[oslot, pl.ds(0, 16)] = c0
        ocbuf[oslot, pl.ds(16, 16)] = c1
        ocbuf[oslot, pl.ds(32, 16)] = c2
        ov, oc = out_copies(r)
        ov.start()
        oc.start()
        return 0

    lax.fori_loop(0, RPW, row_body, 0)
    for t in range(4):
        ov, oc = out_copies(RPW - 4 + t)
        ov.wait()
        oc.wait()


def _emb_block(sv_ref, emb_ref):
    vals_k = lax.slice(sv_ref[...], (0, 1), (RE, KP1))
    sig_i = lax.broadcasted_iota(jnp.int32, (1, 1, NSIG), 2).astype(jnp.float32)
    sig = SIG0 + sig_i * ((SIG1 - SIG0) / (NSIG - 1))
    inv = 0.5 / (sig * sig)
    emb_ref[...] = jnp.exp(-vals_k[:, :, None] * inv)


@jax.jit
def kernel(input_coord):
    x = input_coord.astype(jnp.float32)
    xp = jnp.pad(x, ((0, 0), (0, 5)))
    xt = xp.T

    d2, gm = pl.pallas_call(
        _d2_thr_block,
        grid=(N // R,),
        in_specs=[
            pl.BlockSpec((R, 8), lambda b: (b, 0)),
            pl.BlockSpec((8, N), lambda b: (0, 0)),
        ],
        out_specs=[
            pl.BlockSpec((R, N), lambda b: (b, 0)),
            pl.BlockSpec((R, NG), lambda b: (b, 0)),
        ],
        out_shape=[
            jax.ShapeDtypeStruct((N, N), jnp.float32),
            jax.ShapeDtypeStruct((N, NG), jnp.float32),
        ],
    )(xp, xt)

    thr = pl.pallas_call(
        _thr_block,
        grid=(1,),
        in_specs=[pl.BlockSpec((N, NG), lambda b: (0, 0))],
        out_specs=pl.BlockSpec((N, 16), lambda b: (0, 0)),
        out_shape=jax.ShapeDtypeStruct((N, 16), jnp.float32),
    )(gm)

    mesh = plsc.VectorSubcoreMesh(core_axis_name="c", subcore_axis_name="s")
    sc_call = functools.partial(
        pl.kernel,
        mesh=mesh,
        compiler_params=pltpu.CompilerParams(needs_layout_passes=False),
        out_type=[
            jax.ShapeDtypeStruct((N, 48), jnp.float32),
            jax.ShapeDtypeStruct((N, 48), jnp.int32),
        ],
        scratch_types=[
            pltpu.VMEM((2, N), jnp.float32),
            pltpu.VMEM((RPW, 16), jnp.float32),
            pltpu.VMEM((16 * SLOTS,), jnp.float32),
            pltpu.VMEM((16 * SLOTS,), jnp.int32),
            pltpu.VMEM((4, 48), jnp.float32),
            pltpu.VMEM((4, 48), jnp.int32),
            pltpu.SemaphoreType.DMA((2,)),
            pltpu.SemaphoreType.DMA((4,)),
            pltpu.SemaphoreType.DMA((4,)),
        ],
    )(_sc_topk)
    svals, scols = sc_call(d2, thr)

    emb = pl.pallas_call(
        _emb_block,
        grid=(N // RE,),
        in_specs=[pl.BlockSpec((RE, 48), lambda b: (b, 0))],
        out_specs=pl.BlockSpec((RE, K, NSIG), lambda b: (b, 0, 0)),
        out_shape=jax.ShapeDtypeStruct((N, K, NSIG), jnp.float32),
    )(svals)

    col = lax.slice(scols, (0, 1), (N, KP1)).reshape(N * K)
    row = jnp.repeat(jnp.arange(N, dtype=jnp.int32), K)
    pairs = jnp.stack([row, col], axis=1).astype(jnp.int16)
    return emb.reshape(1, N * K, NSIG), pairs


# speedup vs baseline: 1.2465x; 1.0007x over previous
import functools

import jax
import jax.numpy as jnp
from jax import lax
from jax.experimental import pallas as pl
from jax.experimental.pallas import tpu as pltpu
from jax.experimental.pallas import tpu_sc as plsc

N = 8192
K = 32
KP1 = 33
NSIG = 64
R = 256
RE = 512
NG = 64
GW = N // NG
SLOTS = 24
SIG0, SIG1 = 0.1, 5.0
BIGF = 3.0e38
BIGI = 1 << 30
NW = 32
RPW = N // NW


def _d2_thr_block(xp_ref, xt_ref, d2_ref, gm_ref):
    xb = xp_ref[...]
    xt = xt_ref[...]
    dot = jnp.dot(xb, xt, preferred_element_type=jnp.float32)
    x2r = jnp.sum(xb * xb, axis=1, keepdims=True)
    x2c = jnp.sum(xt * xt, axis=0, keepdims=True)
    d2 = jnp.maximum(x2r + x2c - 2.0 * dot, 0.0)
    colf = lax.broadcasted_iota(jnp.int32, (R, N), 1).astype(jnp.float32)
    d2 = jnp.where(d2 == 0.0, (colf - 8192.0) * (2.0 ** -100), d2)
    d2_ref[...] = d2

    m = d2
    width = N // 2
    while width >= NG:
        m = jnp.minimum(lax.slice(m, (0, 0), (R, width)),
                        lax.slice(m, (0, width), (R, 2 * width)))
        width //= 2
    gm_ref[...] = m


def _thr_block(gm_ref, thr_ref):
    a = gm_ref[...]
    lane = lax.broadcasted_iota(jnp.int32, (N, NG), 1)

    def step(t, carry):
        a, _ = carry
        m = jnp.min(a, axis=1, keepdims=True)
        cand = jnp.where(a == m, lane, BIGI)
        win = jnp.min(cand, axis=1, keepdims=True)
        a = jnp.where(lane == win, BIGF, a)
        return a, m

    _, thr = lax.fori_loop(0, KP1, step, (a, jnp.zeros((N, 1), jnp.float32)))
    thr_ref[...] = jnp.broadcast_to(thr, (N, 16))


def _merge48_16(l0, l1, l2, c0, c1, c2, vs, cs):
    rv = lax.rev(vs, (0,))
    rc = lax.rev(cs, (0,))
    m = l2 <= rv
    nl2 = jnp.where(m, l2, rv)
    nc2 = jnp.where(m, c2, rc)
    nl2, nc2 = plsc.sort_key_val(nl2, nc2)
    rv2 = lax.rev(nl2, (0,))
    rc2 = lax.rev(nc2, (0,))
    m2 = l1 <= rv2
    t1 = jnp.where(m2, l1, rv2)
    tc1 = jnp.where(m2, c1, rc2)
    h1 = jnp.where(m2, rv2, l1)
    hc1 = jnp.where(m2, rc2, c1)
    t1, tc1 = plsc.sort_key_val(t1, tc1)
    h1, hc1 = plsc.sort_key_val(h1, hc1)
    rv3 = lax.rev(t1, (0,))
    rc3 = lax.rev(tc1, (0,))
    m3 = l0 <= rv3
    lo = jnp.where(m3, l0, rv3)
    lc = jnp.where(m3, c0, rc3)
    hi = jnp.where(m3, rv3, l0)
    hc = jnp.where(m3, rc3, c0)
    lo, lc = plsc.sort_key_val(lo, lc)
    hi, hc = plsc.sort_key_val(hi, hc)
    return lo, hi, h1, lc, hc, hc1


def _sc_topk(d2_hbm, thr_hbm, svals_hbm, scols_hbm, rowbuf, thrbuf,
             cbuf, ovbuf, ocbuf, rsem, osem, csem):
    wid = lax.axis_index("s") * 2 + lax.axis_index("c")
    base = wid * RPW
    pltpu.sync_copy(thr_hbm.at[pl.ds(base, RPW), :], thrbuf)

    lane = jnp.arange(16, dtype=jnp.int32)
    lanef = lane.astype(jnp.float32)
    infv = jnp.full((16,), BIGF, jnp.float32) + 0.0 * lanef
    zero16 = jnp.zeros((16,), jnp.int32)

    def in_copy(r):
        return pltpu.make_async_copy(
            d2_hbm.at[pl.ds(base + r, 1), :],
            rowbuf.at[pl.ds(lax.rem(r, 2), 1), pl.ds(0, N)],
            rsem.at[lax.rem(r, 2)])

    def out_copies(r):
        slot = lax.rem(r, 4)
        return (pltpu.make_async_copy(ovbuf.at[pl.ds(slot, 1), :],
                                      svals_hbm.at[pl.ds(base + r, 1), :],
                                      osem.at[slot]),
                pltpu.make_async_copy(ocbuf.at[pl.ds(slot, 1), :],
                                      scols_hbm.at[pl.ds(base + r, 1), :],
                                      csem.at[slot]))

    rowbuf[0, pl.ds(N, 16)] = infv
    rowbuf[1, pl.ds(N, 16)] = infv
    sentinel = jnp.full((16,), N, jnp.int32)
    in_copy(0).start()

    def row_body(r, _):
        @pl.when(r + 1 < RPW)
        def _():
            in_copy(r + 1).start()

        in_copy(r).wait()
        rslot = lax.rem(r, 2)
        rsv = jnp.full((16,), rslot, jnp.int32)
        tv = thrbuf[r, pl.ds(0, 16)]
        for s in range(SLOTS):
            cbuf[pl.ds(s * 16, 16)] = sentinel

        @plsc.parallel_loop(0, N // 16, unroll=16, carry=zero16)
        def cnt_fin(j, cnt):
            v = rowbuf[rslot, pl.ds(j * 16, 16)]
            msk = (v <= tv) & (cnt < SLOTS)
            addr = cnt * 16 + lane
            colv = jnp.full((16,), j * 16, jnp.int32) + lane
            plsc.store_scatter(cbuf, [addr], colv, mask=msk)
            return cnt + msk.astype(jnp.int32)

        maxc = jnp.max(cnt_fin)

        def slot_kv(s):
            cs = cbuf[pl.ds(s * 16, 16)]
            vs = plsc.load_gather(rowbuf, [rsv, cs])
            return plsc.sort_key_val(vs, cs)

        def mbody(s, carry):
            l0, l1, l2, c0, c1, c2 = carry
            vs, cs = slot_kv(s)
            return _merge48_16(l0, l1, l2, c0, c1, c2, vs, cs)

        v0, c0i = slot_kv(0)
        init = (v0, infv, infv, c0i, zero16, zero16)
        l0, l1, l2, c0, c1, c2 = lax.fori_loop(1, maxc, mbody, init)

        @pl.when(r >= 4)
        def _():
            ov, oc = out_copies(r - 4)
            ov.wait()
            oc.wait()

        oslot = lax.rem(r, 4)
        ovbuf[oslot, pl.ds(0, 16)] = l0
        ovbuf[oslot, pl.ds(16, 16)] = l1
        ovbuf[oslot, pl.ds(32, 16)] = l2
        ocbuf[oslot, pl.ds(0, 16)] = c0
        ocbuf[oslot, pl.ds(16, 16)] = c1
        ocbuf[oslot, pl.ds(32, 16)] = c2
        ov, oc = out_copies(r)
        ov.start()
        oc.start()
        return 0

    lax.fori_loop(0, RPW, row_body, 0)
    for t in range(4):
        ov, oc = out_copies(RPW - 4 + t)
        ov.wait()
        oc.wait()


def _emb_block(sv_ref, emb_ref):
    vals_k = lax.slice(sv_ref[...], (0, 1), (RE, KP1))
    sig_i = lax.broadcasted_iota(jnp.int32, (1, 1, NSIG), 2).astype(jnp.float32)
    sig = SIG0 + sig_i * ((SIG1 - SIG0) / (NSIG - 1))
    inv = 0.5 / (sig * sig)
    emb_ref[...] = jnp.exp(-vals_k[:, :, None] * inv)


@jax.jit
def kernel(input_coord):
    x = input_coord.astype(jnp.float32)
    xp = jnp.pad(x, ((0, 0), (0, 5)))
    xt = xp.T

    d2, gm = pl.pallas_call(
        _d2_thr_block,
        grid=(N // R,),
        in_specs=[
            pl.BlockSpec((R, 8), lambda b: (b, 0)),
            pl.BlockSpec((8, N), lambda b: (0, 0)),
        ],
        out_specs=[
            pl.BlockSpec((R, N), lambda b: (b, 0)),
            pl.BlockSpec((R, NG), lambda b: (b, 0)),
        ],
        out_shape=[
            jax.ShapeDtypeStruct((N, N), jnp.float32),
            jax.ShapeDtypeStruct((N, NG), jnp.float32),
        ],
    )(xp, xt)

    thr = pl.pallas_call(
        _thr_block,
        grid=(1,),
        in_specs=[pl.BlockSpec((N, NG), lambda b: (0, 0))],
        out_specs=pl.BlockSpec((N, 16), lambda b: (0, 0)),
        out_shape=jax.ShapeDtypeStruct((N, 16), jnp.float32),
    )(gm)

    mesh = plsc.VectorSubcoreMesh(core_axis_name="c", subcore_axis_name="s")
    sc_call = functools.partial(
        pl.kernel,
        mesh=mesh,
        compiler_params=pltpu.CompilerParams(needs_layout_passes=False),
        out_type=[
            jax.ShapeDtypeStruct((N, 48), jnp.float32),
            jax.ShapeDtypeStruct((N, 48), jnp.int32),
        ],
        scratch_types=[
            pltpu.VMEM((2, N + 16), jnp.float32),
            pltpu.VMEM((RPW, 16), jnp.float32),
            pltpu.VMEM((16 * SLOTS,), jnp.int32),
            pltpu.VMEM((4, 48), jnp.float32),
            pltpu.VMEM((4, 48), jnp.int32),
            pltpu.SemaphoreType.DMA((2,)),
            pltpu.SemaphoreType.DMA((4,)),
            pltpu.SemaphoreType.DMA((4,)),
        ],
    )(_sc_topk)
    svals, scols = sc_call(d2, thr)

    emb = pl.pallas_call(
        _emb_block,
        grid=(N // RE,),
        in_specs=[pl.BlockSpec((RE, 48), lambda b: (b, 0))],
        out_specs=pl.BlockSpec((RE, K, NSIG), lambda b: (b, 0, 0)),
        out_shape=jax.ShapeDtypeStruct((N, K, NSIG), jnp.float32),
    )(svals)

    col = lax.slice(scols, (0, 1), (N, KP1)).reshape(N * K)
    row = jnp.repeat(jnp.arange(N, dtype=jnp.int32), K)
    pairs = jnp.stack([row, col], axis=1).astype(jnp.int16)
    return emb.reshape(1, N * K, NSIG), pairs


# speedup vs baseline: 1.3690x; 1.0982x over previous
import functools

import jax
import jax.numpy as jnp
from jax import lax
from jax.experimental import pallas as pl
from jax.experimental.pallas import tpu as pltpu
from jax.experimental.pallas import tpu_sc as plsc

N = 8192
K = 32
KP1 = 33
NSIG = 64
R = 256
RE = 512
NG = 64
GW = N // NG
SLOTS = 24
SIG0, SIG1 = 0.1, 5.0
BIGF = 3.0e38
BIGI = 1 << 30
NW = 32
RPW = N // NW


def _d2_thr_block(xp_ref, xt_ref, d2_ref, gm_ref):
    xb = xp_ref[...]
    xt = xt_ref[...]
    dot = jnp.dot(xb, xt, preferred_element_type=jnp.float32)
    x2r = jnp.sum(xb * xb, axis=1, keepdims=True)
    x2c = jnp.sum(xt * xt, axis=0, keepdims=True)
    d2 = jnp.maximum(x2r + x2c - 2.0 * dot, 0.0)
    colf = lax.broadcasted_iota(jnp.int32, (R, N), 1).astype(jnp.float32)
    d2 = jnp.where(d2 == 0.0, (colf - 8192.0) * (2.0 ** -100), d2)
    d2_ref[...] = d2

    m = d2
    width = N // 2
    while width >= NG:
        m = jnp.minimum(lax.slice(m, (0, 0), (R, width)),
                        lax.slice(m, (0, width), (R, 2 * width)))
        width //= 2
    gm_ref[...] = m


def _thr_block(gm_ref, thr_ref):
    a = gm_ref[...]
    lane = lax.broadcasted_iota(jnp.int32, (N, NG), 1)

    def step(t, carry):
        a, _ = carry
        m = jnp.min(a, axis=1, keepdims=True)
        cand = jnp.where(a == m, lane, BIGI)
        win = jnp.min(cand, axis=1, keepdims=True)
        a = jnp.where(lane == win, BIGF, a)
        return a, m

    _, thr = lax.fori_loop(0, KP1, step, (a, jnp.zeros((N, 1), jnp.float32)))
    thr_ref[...] = jnp.broadcast_to(thr, (N, 16))


def _merge48_16(l0, l1, l2, c0, c1, c2, vs, cs):
    rv = lax.rev(vs, (0,))
    rc = lax.rev(cs, (0,))
    m = l2 <= rv
    nl2 = jnp.where(m, l2, rv)
    nc2 = jnp.where(m, c2, rc)
    nl2, nc2 = plsc.sort_key_val(nl2, nc2)
    rv2 = lax.rev(nl2, (0,))
    rc2 = lax.rev(nc2, (0,))
    m2 = l1 <= rv2
    t1 = jnp.where(m2, l1, rv2)
    tc1 = jnp.where(m2, c1, rc2)
    h1 = jnp.where(m2, rv2, l1)
    hc1 = jnp.where(m2, rc2, c1)
    t1, tc1 = plsc.sort_key_val(t1, tc1)
    h1, hc1 = plsc.sort_key_val(h1, hc1)
    rv3 = lax.rev(t1, (0,))
    rc3 = lax.rev(tc1, (0,))
    m3 = l0 <= rv3
    lo = jnp.where(m3, l0, rv3)
    lc = jnp.where(m3, c0, rc3)
    hi = jnp.where(m3, rv3, l0)
    hc = jnp.where(m3, rc3, c0)
    lo, lc = plsc.sort_key_val(lo, lc)
    hi, hc = plsc.sort_key_val(hi, hc)
    return lo, hi, h1, lc, hc, hc1


def _sc_topk(d2_hbm, thr_hbm, svals_hbm, scols_hbm, rowbuf, thrbuf,
             cbuf, ovbuf, ocbuf, rsem, osem, csem):
    wid = lax.axis_index("s") * 2 + lax.axis_index("c")
    base = wid * RPW
    pltpu.sync_copy(thr_hbm.at[pl.ds(base, RPW), :], thrbuf)

    lane = jnp.arange(16, dtype=jnp.int32)
    lanef = lane.astype(jnp.float32)
    infv = jnp.full((16,), BIGF, jnp.float32) + 0.0 * lanef
    zero16 = jnp.zeros((16,), jnp.int32)

    def in_copy(r):
        return pltpu.make_async_copy(
            d2_hbm.at[pl.ds(base + r, 1), :],
            rowbuf.at[pl.ds(lax.rem(r, 2), 1), pl.ds(0, N)],
            rsem.at[lax.rem(r, 2)])

    def out_copies(r):
        slot = lax.rem(r, 4)
        return (pltpu.make_async_copy(ovbuf.at[pl.ds(slot, 1), :],
                                      svals_hbm.at[pl.ds(base + r, 1), :],
                                      osem.at[slot]),
                pltpu.make_async_copy(ocbuf.at[pl.ds(slot, 1), :],
                                      scols_hbm.at[pl.ds(base + r, 1), :],
                                      csem.at[slot]))

    rowbuf[0, pl.ds(N, 16)] = infv
    rowbuf[1, pl.ds(N, 16)] = infv
    sentinel = jnp.full((16,), N, jnp.int32)
    in_copy(0).start()

    def row_body(r, _):
        @pl.when(r + 1 < RPW)
        def _():
            in_copy(r + 1).start()

        in_copy(r).wait()
        rslot = lax.rem(r, 2)
        rsv = jnp.full((16,), rslot, jnp.int32)
        tv = thrbuf[r, pl.ds(0, 16)]
        for s in range(SLOTS):
            cbuf[pl.ds(s * 16, 16)] = sentinel

        @plsc.parallel_loop(0, N // 16, unroll=16, carry=zero16)
        def cnt_fin(j, cnt):
            v = rowbuf[rslot, pl.ds(j * 16, 16)]
            msk = v <= tv
            addr = jnp.minimum(cnt, SLOTS - 1) * 16 + lane
            colv = jnp.full((16,), j * 16, jnp.int32) + lane
            plsc.store_scatter(cbuf, [addr], colv, mask=msk)
            return cnt + msk.astype(jnp.int32)

        maxc = jnp.minimum(jnp.max(cnt_fin), SLOTS)

        def slot_kv(s):
            cs = cbuf[pl.ds(s * 16, 16)]
            vs = plsc.load_gather(rowbuf, [rsv, cs])
            return plsc.sort_key_val(vs, cs)

        def mbody(s, carry):
            l0, l1, l2, c0, c1, c2 = carry
            vs, cs = slot_kv(s)
            return _merge48_16(l0, l1, l2, c0, c1, c2, vs, cs)

        v0, c0i = slot_kv(0)
        init = (v0, infv, infv, c0i, zero16, zero16)
        l0, l1, l2, c0, c1, c2 = lax.fori_loop(1, maxc, mbody, init)

        @pl.when(r >= 4)
        def _():
            ov, oc = out_copies(r - 4)
            ov.wait()
            oc.wait()

        oslot = lax.rem(r, 4)
        ovbuf[oslot, pl.ds(0, 16)] = l0
        ovbuf[oslot, pl.ds(16, 16)] = l1
        ovbuf[oslot, pl.ds(32, 16)] = l2
        ocbuf[oslot, pl.ds(0, 16)] = c0
        ocbuf[oslot, pl.ds(16, 16)] = c1
        ocbuf[oslot, pl.ds(32, 16)] = c2
        ov, oc = out_copies(r)
        ov.start()
        oc.start()
        return 0

    lax.fori_loop(0, RPW, row_body, 0)
    for t in range(4):
        ov, oc = out_copies(RPW - 4 + t)
        ov.wait()
        oc.wait()


def _emb_block(sv_ref, emb_ref):
    vals_k = lax.slice(sv_ref[...], (0, 1), (RE, KP1))
    sig_i = lax.broadcasted_iota(jnp.int32, (1, 1, NSIG), 2).astype(jnp.float32)
    sig = SIG0 + sig_i * ((SIG1 - SIG0) / (NSIG - 1))
    inv = 0.5 / (sig * sig)
    emb_ref[...] = jnp.exp(-vals_k[:, :, None] * inv)


@jax.jit
def kernel(input_coord):
    x = input_coord.astype(jnp.float32)
    xp = jnp.pad(x, ((0, 0), (0, 5)))
    xt = xp.T

    d2, gm = pl.pallas_call(
        _d2_thr_block,
        grid=(N // R,),
        in_specs=[
            pl.BlockSpec((R, 8), lambda b: (b, 0)),
            pl.BlockSpec((8, N), lambda b: (0, 0)),
        ],
        out_specs=[
            pl.BlockSpec((R, N), lambda b: (b, 0)),
            pl.BlockSpec((R, NG), lambda b: (b, 0)),
        ],
        out_shape=[
            jax.ShapeDtypeStruct((N, N), jnp.float32),
            jax.ShapeDtypeStruct((N, NG), jnp.float32),
        ],
    )(xp, xt)

    thr = pl.pallas_call(
        _thr_block,
        grid=(1,),
        in_specs=[pl.BlockSpec((N, NG), lambda b: (0, 0))],
        out_specs=pl.BlockSpec((N, 16), lambda b: (0, 0)),
        out_shape=jax.ShapeDtypeStruct((N, 16), jnp.float32),
    )(gm)

    mesh = plsc.VectorSubcoreMesh(core_axis_name="c", subcore_axis_name="s")
    sc_call = functools.partial(
        pl.kernel,
        mesh=mesh,
        compiler_params=pltpu.CompilerParams(needs_layout_passes=False),
        out_type=[
            jax.ShapeDtypeStruct((N, 48), jnp.float32),
            jax.ShapeDtypeStruct((N, 48), jnp.int32),
        ],
        scratch_types=[
            pltpu.VMEM((2, N + 16), jnp.float32),
            pltpu.VMEM((RPW, 16), jnp.float32),
            pltpu.VMEM((16 * SLOTS,), jnp.int32),
            pltpu.VMEM((4, 48), jnp.float32),
            pltpu.VMEM((4, 48), jnp.int32),
            pltpu.SemaphoreType.DMA((2,)),
            pltpu.SemaphoreType.DMA((4,)),
            pltpu.SemaphoreType.DMA((4,)),
        ],
    )(_sc_topk)
    svals, scols = sc_call(d2, thr)

    emb = pl.pallas_call(
        _emb_block,
        grid=(N // RE,),
        in_specs=[pl.BlockSpec((RE, 48), lambda b: (b, 0))],
        out_specs=pl.BlockSpec((RE, K, NSIG), lambda b: (b, 0, 0)),
        out_shape=jax.ShapeDtypeStruct((N, K, NSIG), jnp.float32),
    )(svals)

    col = lax.slice(scols, (0, 1), (N, KP1)).reshape(N * K)
    row = jnp.repeat(jnp.arange(N, dtype=jnp.int32), K)
    pairs = jnp.stack([row, col], axis=1).astype(jnp.int16)
    return emb.reshape(1, N * K, NSIG), pairs
